# baseline (speedup 1.0000x reference)
import functools

import jax
import jax.numpy as jnp
from jax import lax
from jax.experimental import pallas as pl
from jax.experimental.pallas import tpu as pltpu
from jax.experimental.pallas import tpu_sc as plsc

N, E, D, C, G, H = 10000, 320000, 128, 32, 64, 64

NC, NS = 2, 16
NW = NC * NS
CH = 256
NCHUNK = E // CH
CPT = NCHUNK // NW
NTAIL = NCHUNK - CPT * NW
CPTF = NCHUNK // NS
NTAILF = NCHUNK - CPTF * NS
HH = H // NC
ACC_ROWS = 10112
RPT = ACC_ROWS // NS

_mesh = plsc.VectorSubcoreMesh(core_axis_name="c", subcore_axis_name="s")
_sc_params = pltpu.CompilerParams(use_tc_tiling_on_sc=False,
                                 needs_layout_passes=False)


@functools.partial(
    pl.kernel,
    out_type=jax.ShapeDtypeStruct((NC * ACC_ROWS,), jnp.float32),
    mesh=_mesh,
    compiler_params=_sc_params,
    scratch_types=[
        pltpu.VMEM((CPT + 1, CH), jnp.int32),
        pltpu.VMEM((CH, 16), jnp.float32),
        pltpu.VMEM((8, 16), jnp.float32),
        pltpu.VMEM((640, 16), jnp.float32),
        pltpu.VMEM((640,), jnp.float32),
        pltpu.VMEM_SHARED((ACC_ROWS, 16), jnp.float32),
    ],
)
def _sc_degree(e3d_hbm, out_hbm, idx_v, ones_v, z8, hist_v, deg_v, acc):
    c = lax.axis_index("c")
    s = lax.axis_index("s")
    wid = c * NS + s
    for r in range(8):
        z8[r, :] = jnp.zeros((16,), jnp.float32)
    for r in range(CH):
        ones_v[r, :] = jnp.ones((16,), jnp.float32)
    zb = s * RPT
    lax.fori_loop(
        0, RPT // 8,
        lambda j, _: (pltpu.sync_copy(z8, acc.at[pl.ds(zb + j * 8, 8)]), _)[1],
        None)
    pltpu.sync_copy(e3d_hbm.at[1, pl.ds(wid * CPT, CPT)], idx_v.at[pl.ds(0, CPT)])

    @pl.when(wid < NTAIL)
    def _tail_load():
        pltpu.sync_copy(e3d_hbm.at[1, pl.ds(NW * CPT + wid, 1)],
                        idx_v.at[pl.ds(CPT, 1)])

    plsc.subcore_barrier()
    lax.fori_loop(
        0, CPT,
        lambda j, _: (pltpu.sync_copy(ones_v, acc.at[idx_v.at[j]], add=True), _)[1],
        None)

    @pl.when(wid < NTAIL)
    def _tail_scatter():
        pltpu.sync_copy(ones_v, acc.at[idx_v.at[CPT]], add=True)

    plsc.subcore_barrier()
    pltpu.sync_copy(acc.at[pl.ds(s * RPT, RPT)], hist_v.at[pl.ds(0, RPT)])
    col0 = jnp.zeros((16,), jnp.int32)

    def _xpose(k, _):
        rows = k * 16 + lax.iota(jnp.int32, 16)
        deg_v[pl.ds(k * 16, 16)] = plsc.load_gather(hist_v, [rows, col0])
        return _

    lax.fori_loop(0, 640 // 16, _xpose, None)
    pltpu.sync_copy(deg_v.at[pl.ds(0, RPT)],
                    out_hbm.at[pl.ds(c * ACC_ROWS + s * RPT, RPT)])


@functools.partial(
    pl.kernel,
    out_type=jax.ShapeDtypeStruct((NC, ACC_ROWS, HH), jnp.float32),
    mesh=_mesh,
    compiler_params=_sc_params,
    scratch_types=[
        pltpu.VMEM((CPTF + 1, CH), jnp.int32),
        pltpu.VMEM((CPTF + 1, CH), jnp.int32),
        pltpu.VMEM((CH, HH), jnp.float32),
        pltpu.VMEM((CH, HH), jnp.float32),
        pltpu.VMEM((8, HH), jnp.float32),
        pltpu.VMEM_SHARED((ACC_ROWS, HH), jnp.float32),
        pltpu.VMEM_SHARED((ACC_ROWS, HH), jnp.float32),
        pltpu.SemaphoreType.DMA,
        pltpu.SemaphoreType.DMA,
    ],
)
def _sc_aggregate(e3d_hbm, tbl_hbm, out_hbm,
                  src_v, dst_v, rows_a, rows_b, z8, acc, tbl_s, sem_a, sem_b):
    c = lax.axis_index("c")
    s = lax.axis_index("s")
    for r in range(8):
        for q in range(HH // 16):
            z8[r, pl.ds(q * 16, 16)] = jnp.zeros((16,), jnp.float32)
    zb = s * RPT
    pltpu.sync_copy(tbl_hbm.at[c, pl.ds(zb, RPT)], tbl_s.at[pl.ds(zb, RPT)])
    lax.fori_loop(
        0, RPT // 8,
        lambda j, _: (pltpu.sync_copy(z8, acc.at[pl.ds(zb + j * 8, 8)]), _)[1],
        None)
    base = s * CPTF
    pltpu.sync_copy(e3d_hbm.at[0, pl.ds(base, CPTF)], src_v.at[pl.ds(0, CPTF)])
    pltpu.sync_copy(e3d_hbm.at[1, pl.ds(base, CPTF)], dst_v.at[pl.ds(0, CPTF)])

    @pl.when(s < NTAILF)
    def _tail_load():
        pltpu.sync_copy(e3d_hbm.at[0, pl.ds(NS * CPTF + s, 1)],
                        src_v.at[pl.ds(CPTF, 1)])
        pltpu.sync_copy(e3d_hbm.at[1, pl.ds(NS * CPTF + s, 1)],
                        dst_v.at[pl.ds(CPTF, 1)])

    plsc.subcore_barrier()

    bufs = ((rows_a, sem_a), (rows_b, sem_b))
    pltpu.async_copy(tbl_s.at[src_v.at[0]], rows_a, sem_a)

    def group(g, _):
        for b in range(2):
            j = 2 * g + b
            buf, sem = bufs[b]
            nbuf, nsem = bufs[1 - b]
            if b == 0:
                pltpu.async_copy(tbl_s.at[src_v.at[j + 1]], nbuf, nsem)
            else:
                @pl.when(j + 1 < CPTF)
                def _start():
                    pltpu.async_copy(tbl_s.at[src_v.at[j + 1]], nbuf, nsem)
            pltpu.make_async_copy(tbl_s.at[src_v.at[j]], buf, sem).wait()
            pltpu.sync_copy(buf, acc.at[dst_v.at[j]], add=True)
        return _

    lax.fori_loop(0, CPTF // 2, group, None)

    @pl.when(s < NTAILF)
    def _tail_chunk():
        pltpu.async_copy(tbl_s.at[src_v.at[CPTF]], rows_b, sem_b).wait()
        pltpu.sync_copy(rows_b, acc.at[dst_v.at[CPTF]], add=True)

    plsc.subcore_barrier()
    pltpu.sync_copy(acc.at[pl.ds(s * RPT, RPT)],
                    out_hbm.at[c, pl.ds(s * RPT, RPT)])


def _tc0_body(x_ref, w1_ref, m_ref):
    m_ref[...] = jnp.dot(x_ref[...], w1_ref[...],
                         preferred_element_type=jnp.float32)


def _split_store(t_ref, v):
    t_ref[0, 0:N, :] = v[:, 0:HH]
    t_ref[1, 0:N, :] = v[:, HH:H]
    t_ref[0, N:ACC_ROWS, :] = jnp.zeros((ACC_ROWS - N, HH), jnp.float32)
    t_ref[1, N:ACC_ROWS, :] = jnp.zeros((ACC_ROWS - N, HH), jnp.float32)


def _merge(a_ref, t_ref=None):
    agg = jnp.concatenate([a_ref[0, 0:N, :], a_ref[1, 0:N, :]], axis=1)
    if t_ref is None:
        return agg
    return agg + jnp.concatenate([t_ref[0, 0:N, :], t_ref[1, 0:N, :]], axis=1)


def _tc1_body(m_ref, degp_ref, t1_ref, dinv_ref):
    deg = degp_ref[0:N] + degp_ref[ACC_ROWS:ACC_ROWS + N] + 1.0
    dinv = lax.rsqrt(deg)
    _split_store(t1_ref, m_ref[...] * dinv[:, None])
    dinv_ref[...] = dinv[:, None]


def _tc2_body(aggp_ref, t1_ref, dinv_ref, b1_ref, w2_ref, t2_ref):
    dinv = dinv_ref[...]
    h1 = jax.nn.relu(dinv * _merge(aggp_ref, t1_ref) + b1_ref[...])
    m2 = jnp.dot(h1, w2_ref[...], preferred_element_type=jnp.float32)
    _split_store(t2_ref, m2 * dinv)


def _tc3_body(aggp_ref, t2_ref, dinv_ref, b2_ref, batch_ref, cfg_ref,
              fc1w_ref, fc1b_ref, fc2w_ref, fc2b_ref, out_ref):
    h2 = jax.nn.relu(dinv_ref[...] * _merge(aggp_ref, t2_ref) + b2_ref[...])
    gids = lax.broadcasted_iota(jnp.int32, (N, G), 1)
    p = (batch_ref[...] == gids).astype(jnp.float32)
    sums = lax.dot_general(p, h2, (((0,), (0,)), ((), ())),
                           preferred_element_type=jnp.float32)
    cnt = jnp.sum(p, axis=0)
    pooled = sums / jnp.maximum(cnt, 1.0)[:, None]
    z = jnp.concatenate([pooled, cfg_ref[...]], axis=1)
    z = jax.nn.relu(jnp.dot(z, fc1w_ref[...],
                            preferred_element_type=jnp.float32) + fc1b_ref[...])
    out_ref[...] = jnp.dot(z, fc2w_ref[...],
                           preferred_element_type=jnp.float32) + fc2b_ref[...]


def _vmem_call(body, n_in, out_shapes):
    return pl.pallas_call(
        body,
        in_specs=[pl.BlockSpec(memory_space=pltpu.VMEM) for _ in range(n_in)],
        out_specs=tuple(pl.BlockSpec(memory_space=pltpu.VMEM)
                        for _ in out_shapes) if len(out_shapes) > 1
        else pl.BlockSpec(memory_space=pltpu.VMEM),
        out_shape=out_shapes if len(out_shapes) > 1 else out_shapes[0],
    )


def kernel(x, edge_index, config, batch, W1, b1, W2, b2,
           fc1_W, fc1_b, fc2_W, fc2_b):
    e3d = edge_index.astype(jnp.int32).reshape(2, NCHUNK, CH)

    degp = _sc_degree(e3d)
    m1 = _vmem_call(
        _tc0_body, 2,
        (jax.ShapeDtypeStruct((N, H), jnp.float32),),
    )(x, W1)

    t1, dinv = _vmem_call(
        _tc1_body, 2,
        (jax.ShapeDtypeStruct((NC, ACC_ROWS, HH), jnp.float32),
         jax.ShapeDtypeStruct((N, 1), jnp.float32)),
    )(m1, degp)

    aggp1 = _sc_aggregate(e3d, t1)

    t2 = _vmem_call(
        _tc2_body, 5,
        (jax.ShapeDtypeStruct((NC, ACC_ROWS, HH), jnp.float32),),
    )(aggp1, t1, dinv, b1.reshape(1, H), W2)

    aggp2 = _sc_aggregate(e3d, t2)

    out = _vmem_call(
        _tc3_body, 10,
        (jax.ShapeDtypeStruct((G, 1), jnp.float32),),
    )(aggp2, t2, dinv, b2.reshape(1, H), batch.astype(jnp.int32).reshape(N, 1),
      config, fc1_W, fc1_b.reshape(1, H), fc2_W, fc2_b.reshape(1, 1))
    return out


# speedup vs baseline: 1.0070x; 1.0070x over previous
import functools

import jax
import jax.numpy as jnp
from jax import lax
from jax.experimental import pallas as pl
from jax.experimental.pallas import tpu as pltpu
from jax.experimental.pallas import tpu_sc as plsc

N, E, D, C, G, H = 10000, 320000, 128, 32, 64, 64

NC, NS = 2, 16
NW = NC * NS
CH = 128
NCHUNK = E // CH
CPT = NCHUNK // NW
NTAIL = NCHUNK - CPT * NW
CPTF = NCHUNK // NS
NTAILF = NCHUNK - CPTF * NS
HH = H // NC
ACC_ROWS = 10112
RPT = ACC_ROWS // NS

_mesh = plsc.VectorSubcoreMesh(core_axis_name="c", subcore_axis_name="s")
_sc_params = pltpu.CompilerParams(use_tc_tiling_on_sc=False,
                                 needs_layout_passes=False)


@functools.partial(
    pl.kernel,
    out_type=jax.ShapeDtypeStruct((NC * ACC_ROWS,), jnp.float32),
    mesh=_mesh,
    compiler_params=_sc_params,
    scratch_types=[
        pltpu.VMEM((CPT + 1, CH), jnp.int32),
        pltpu.VMEM((CH, 16), jnp.float32),
        pltpu.VMEM((8, 16), jnp.float32),
        pltpu.VMEM((640, 16), jnp.float32),
        pltpu.VMEM((640,), jnp.float32),
        pltpu.VMEM_SHARED((ACC_ROWS, 16), jnp.float32),
    ],
)
def _sc_degree(e3d_hbm, out_hbm, idx_v, ones_v, z8, hist_v, deg_v, acc):
    c = lax.axis_index("c")
    s = lax.axis_index("s")
    wid = c * NS + s
    for r in range(8):
        z8[r, :] = jnp.zeros((16,), jnp.float32)
    for r in range(CH):
        ones_v[r, :] = jnp.ones((16,), jnp.float32)
    zb = s * RPT
    lax.fori_loop(
        0, RPT // 8,
        lambda j, _: (pltpu.sync_copy(z8, acc.at[pl.ds(zb + j * 8, 8)]), _)[1],
        None)
    pltpu.sync_copy(e3d_hbm.at[1, pl.ds(wid * CPT, CPT)], idx_v.at[pl.ds(0, CPT)])

    @pl.when(wid < NTAIL)
    def _tail_load():
        pltpu.sync_copy(e3d_hbm.at[1, pl.ds(NW * CPT + wid, 1)],
                        idx_v.at[pl.ds(CPT, 1)])

    plsc.subcore_barrier()
    lax.fori_loop(
        0, CPT,
        lambda j, _: (pltpu.sync_copy(ones_v, acc.at[idx_v.at[j]], add=True), _)[1],
        None)

    @pl.when(wid < NTAIL)
    def _tail_scatter():
        pltpu.sync_copy(ones_v, acc.at[idx_v.at[CPT]], add=True)

    plsc.subcore_barrier()
    pltpu.sync_copy(acc.at[pl.ds(s * RPT, RPT)], hist_v.at[pl.ds(0, RPT)])
    col0 = jnp.zeros((16,), jnp.int32)

    def _xpose(k, _):
        rows = k * 16 + lax.iota(jnp.int32, 16)
        deg_v[pl.ds(k * 16, 16)] = plsc.load_gather(hist_v, [rows, col0])
        return _

    lax.fori_loop(0, 640 // 16, _xpose, None)
    pltpu.sync_copy(deg_v.at[pl.ds(0, RPT)],
                    out_hbm.at[pl.ds(c * ACC_ROWS + s * RPT, RPT)])


@functools.partial(
    pl.kernel,
    out_type=jax.ShapeDtypeStruct((NC, ACC_ROWS, HH), jnp.float32),
    mesh=_mesh,
    compiler_params=_sc_params,
    scratch_types=[
        pltpu.VMEM((CPTF + 1, CH), jnp.int32),
        pltpu.VMEM((CPTF + 1, CH), jnp.int32),
        pltpu.VMEM((CH, HH), jnp.float32),
        pltpu.VMEM((CH, HH), jnp.float32),
        pltpu.VMEM((8, HH), jnp.float32),
        pltpu.VMEM_SHARED((ACC_ROWS, HH), jnp.float32),
        pltpu.VMEM_SHARED((ACC_ROWS, HH), jnp.float32),
        pltpu.SemaphoreType.DMA,
        pltpu.SemaphoreType.DMA,
    ],
)
def _sc_aggregate(e3d_hbm, tbl_hbm, out_hbm,
                  src_v, dst_v, rows_a, rows_b, z8, acc, tbl_s, sem_a, sem_b):
    c = lax.axis_index("c")
    s = lax.axis_index("s")
    for r in range(8):
        for q in range(HH // 16):
            z8[r, pl.ds(q * 16, 16)] = jnp.zeros((16,), jnp.float32)
    zb = s * RPT
    pltpu.sync_copy(tbl_hbm.at[c, pl.ds(zb, RPT)], tbl_s.at[pl.ds(zb, RPT)])
    lax.fori_loop(
        0, RPT // 8,
        lambda j, _: (pltpu.sync_copy(z8, acc.at[pl.ds(zb + j * 8, 8)]), _)[1],
        None)
    base = s * CPTF
    pltpu.sync_copy(e3d_hbm.at[0, pl.ds(base, CPTF)], src_v.at[pl.ds(0, CPTF)])
    pltpu.sync_copy(e3d_hbm.at[1, pl.ds(base, CPTF)], dst_v.at[pl.ds(0, CPTF)])

    @pl.when(s < NTAILF)
    def _tail_load():
        pltpu.sync_copy(e3d_hbm.at[0, pl.ds(NS * CPTF + s, 1)],
                        src_v.at[pl.ds(CPTF, 1)])
        pltpu.sync_copy(e3d_hbm.at[1, pl.ds(NS * CPTF + s, 1)],
                        dst_v.at[pl.ds(CPTF, 1)])

    plsc.subcore_barrier()

    bufs = ((rows_a, sem_a), (rows_b, sem_b))
    pltpu.async_copy(tbl_s.at[src_v.at[0]], rows_a, sem_a)

    def group(g, _):
        for b in range(2):
            j = 2 * g + b
            buf, sem = bufs[b]
            nbuf, nsem = bufs[1 - b]
            if b == 0:
                pltpu.async_copy(tbl_s.at[src_v.at[j + 1]], nbuf, nsem)
            else:
                @pl.when(j + 1 < CPTF)
                def _start():
                    pltpu.async_copy(tbl_s.at[src_v.at[j + 1]], nbuf, nsem)
            pltpu.make_async_copy(tbl_s.at[src_v.at[j]], buf, sem).wait()
            pltpu.sync_copy(buf, acc.at[dst_v.at[j]], add=True)
        return _

    lax.fori_loop(0, CPTF // 2, group, None)

    @pl.when(s < NTAILF)
    def _tail_chunk():
        pltpu.async_copy(tbl_s.at[src_v.at[CPTF]], rows_b, sem_b).wait()
        pltpu.sync_copy(rows_b, acc.at[dst_v.at[CPTF]], add=True)

    plsc.subcore_barrier()
    pltpu.sync_copy(acc.at[pl.ds(s * RPT, RPT)],
                    out_hbm.at[c, pl.ds(s * RPT, RPT)])


def _tc0_body(x_ref, w1_ref, m_ref):
    m_ref[...] = jnp.dot(x_ref[...], w1_ref[...],
                         preferred_element_type=jnp.float32)


def _split_store(t_ref, v):
    t_ref[0, 0:N, :] = v[:, 0:HH]
    t_ref[1, 0:N, :] = v[:, HH:H]
    t_ref[0, N:ACC_ROWS, :] = jnp.zeros((ACC_ROWS - N, HH), jnp.float32)
    t_ref[1, N:ACC_ROWS, :] = jnp.zeros((ACC_ROWS - N, HH), jnp.float32)


def _merge(a_ref, t_ref=None):
    agg = jnp.concatenate([a_ref[0, 0:N, :], a_ref[1, 0:N, :]], axis=1)
    if t_ref is None:
        return agg
    return agg + jnp.concatenate([t_ref[0, 0:N, :], t_ref[1, 0:N, :]], axis=1)


def _tc1_body(m_ref, degp_ref, t1_ref, dinv_ref):
    deg = degp_ref[0:N] + degp_ref[ACC_ROWS:ACC_ROWS + N] + 1.0
    dinv = lax.rsqrt(deg)
    _split_store(t1_ref, m_ref[...] * dinv[:, None])
    dinv_ref[...] = dinv[:, None]


def _tc2_body(aggp_ref, t1_ref, dinv_ref, b1_ref, w2_ref, t2_ref):
    dinv = dinv_ref[...]
    h1 = jax.nn.relu(dinv * _merge(aggp_ref, t1_ref) + b1_ref[...])
    m2 = jnp.dot(h1, w2_ref[...], preferred_element_type=jnp.float32)
    _split_store(t2_ref, m2 * dinv)


def _tc3_body(aggp_ref, t2_ref, dinv_ref, b2_ref, batch_ref, cfg_ref,
              fc1w_ref, fc1b_ref, fc2w_ref, fc2b_ref, out_ref):
    h2 = jax.nn.relu(dinv_ref[...] * _merge(aggp_ref, t2_ref) + b2_ref[...])
    gids = lax.broadcasted_iota(jnp.int32, (N, G), 1)
    p = (batch_ref[...] == gids).astype(jnp.float32)
    sums = lax.dot_general(p, h2, (((0,), (0,)), ((), ())),
                           preferred_element_type=jnp.float32)
    cnt = jnp.sum(p, axis=0)
    pooled = sums / jnp.maximum(cnt, 1.0)[:, None]
    z = jnp.concatenate([pooled, cfg_ref[...]], axis=1)
    z = jax.nn.relu(jnp.dot(z, fc1w_ref[...],
                            preferred_element_type=jnp.float32) + fc1b_ref[...])
    out_ref[...] = jnp.dot(z, fc2w_ref[...],
                           preferred_element_type=jnp.float32) + fc2b_ref[...]


def _vmem_call(body, n_in, out_shapes):
    return pl.pallas_call(
        body,
        in_specs=[pl.BlockSpec(memory_space=pltpu.VMEM) for _ in range(n_in)],
        out_specs=tuple(pl.BlockSpec(memory_space=pltpu.VMEM)
                        for _ in out_shapes) if len(out_shapes) > 1
        else pl.BlockSpec(memory_space=pltpu.VMEM),
        out_shape=out_shapes if len(out_shapes) > 1 else out_shapes[0],
    )


def kernel(x, edge_index, config, batch, W1, b1, W2, b2,
           fc1_W, fc1_b, fc2_W, fc2_b):
    e3d = edge_index.astype(jnp.int32).reshape(2, NCHUNK, CH)

    degp = _sc_degree(e3d)
    m1 = _vmem_call(
        _tc0_body, 2,
        (jax.ShapeDtypeStruct((N, H), jnp.float32),),
    )(x, W1)

    t1, dinv = _vmem_call(
        _tc1_body, 2,
        (jax.ShapeDtypeStruct((NC, ACC_ROWS, HH), jnp.float32),
         jax.ShapeDtypeStruct((N, 1), jnp.float32)),
    )(m1, degp)

    aggp1 = _sc_aggregate(e3d, t1)

    t2 = _vmem_call(
        _tc2_body, 5,
        (jax.ShapeDtypeStruct((NC, ACC_ROWS, HH), jnp.float32),),
    )(aggp1, t1, dinv, b1.reshape(1, H), W2)

    aggp2 = _sc_aggregate(e3d, t2)

    out = _vmem_call(
        _tc3_body, 10,
        (jax.ShapeDtypeStruct((G, 1), jnp.float32),),
    )(aggp2, t2, dinv, b2.reshape(1, H), batch.astype(jnp.int32).reshape(N, 1),
      config, fc1_W, fc1_b.reshape(1, H), fc2_W, fc2_b.reshape(1, 1))
    return out


# speedup vs baseline: 1.0073x; 1.0003x over previous
import functools

import jax
import jax.numpy as jnp
from jax import lax
from jax.experimental import pallas as pl
from jax.experimental.pallas import tpu as pltpu
from jax.experimental.pallas import tpu_sc as plsc

N, E, D, C, G, H = 10000, 320000, 128, 32, 64, 64

NC, NS = 2, 16
NW = NC * NS
CH = 128
NCHUNK = E // CH
CPT = NCHUNK // NW
NTAIL = NCHUNK - CPT * NW
CPTF = NCHUNK // NS
NTAILF = NCHUNK - CPTF * NS
HH = H // NC
ACC_ROWS = 10112
RPT = ACC_ROWS // NS

_mesh = plsc.VectorSubcoreMesh(core_axis_name="c", subcore_axis_name="s")
_sc_params = pltpu.CompilerParams(use_tc_tiling_on_sc=False,
                                 needs_layout_passes=False)


@functools.partial(
    pl.kernel,
    out_type=jax.ShapeDtypeStruct((NC * ACC_ROWS,), jnp.float32),
    mesh=_mesh,
    compiler_params=_sc_params,
    scratch_types=[
        pltpu.VMEM((CPT + 1, CH), jnp.int32),
        pltpu.VMEM((CH, 16), jnp.float32),
        pltpu.VMEM((8, 16), jnp.float32),
        pltpu.VMEM((640, 16), jnp.float32),
        pltpu.VMEM((640,), jnp.float32),
        pltpu.VMEM_SHARED((ACC_ROWS, 16), jnp.float32),
    ],
)
def _sc_degree(e3d_hbm, out_hbm, idx_v, ones_v, z8, hist_v, deg_v, acc):
    c = lax.axis_index("c")
    s = lax.axis_index("s")
    wid = c * NS + s
    for r in range(8):
        z8[r, :] = jnp.zeros((16,), jnp.float32)
    for r in range(CH):
        ones_v[r, :] = jnp.ones((16,), jnp.float32)
    zb = s * RPT
    lax.fori_loop(
        0, RPT // 8,
        lambda j, _: (pltpu.sync_copy(z8, acc.at[pl.ds(zb + j * 8, 8)]), _)[1],
        None)
    pltpu.sync_copy(e3d_hbm.at[1, pl.ds(wid * CPT, CPT)], idx_v.at[pl.ds(0, CPT)])

    @pl.when(wid < NTAIL)
    def _tail_load():
        pltpu.sync_copy(e3d_hbm.at[1, pl.ds(NW * CPT + wid, 1)],
                        idx_v.at[pl.ds(CPT, 1)])

    plsc.subcore_barrier()
    lax.fori_loop(
        0, CPT,
        lambda j, _: (pltpu.sync_copy(ones_v, acc.at[idx_v.at[j]], add=True), _)[1],
        None)

    @pl.when(wid < NTAIL)
    def _tail_scatter():
        pltpu.sync_copy(ones_v, acc.at[idx_v.at[CPT]], add=True)

    plsc.subcore_barrier()
    pltpu.sync_copy(acc.at[pl.ds(s * RPT, RPT)], hist_v.at[pl.ds(0, RPT)])
    col0 = jnp.zeros((16,), jnp.int32)

    def _xpose(k, _):
        rows = k * 16 + lax.iota(jnp.int32, 16)
        deg_v[pl.ds(k * 16, 16)] = plsc.load_gather(hist_v, [rows, col0])
        return _

    lax.fori_loop(0, 640 // 16, _xpose, None)
    pltpu.sync_copy(deg_v.at[pl.ds(0, RPT)],
                    out_hbm.at[pl.ds(c * ACC_ROWS + s * RPT, RPT)])


@functools.partial(
    pl.kernel,
    out_type=jax.ShapeDtypeStruct((NC, ACC_ROWS, HH), jnp.float32),
    mesh=_mesh,
    compiler_params=_sc_params,
    scratch_types=[
        pltpu.VMEM((CPTF + 1, CH), jnp.int32),
        pltpu.VMEM((CPTF + 1, CH), jnp.int32),
        pltpu.VMEM((CH, HH), jnp.float32),
        pltpu.VMEM((CH, HH), jnp.float32),
        pltpu.VMEM((8, HH), jnp.float32),
        pltpu.VMEM_SHARED((ACC_ROWS, HH), jnp.float32),
        pltpu.VMEM_SHARED((ACC_ROWS, HH), jnp.float32),
        pltpu.SemaphoreType.DMA,
        pltpu.SemaphoreType.DMA,
        pltpu.SemaphoreType.DMA,
        pltpu.SemaphoreType.DMA,
    ],
)
def _sc_aggregate(e3d_hbm, tbl_hbm, out_hbm,
                  src_v, dst_v, rows_a, rows_b, z8, acc, tbl_s,
                  sem_a, sem_b, sem_sa, sem_sb):
    c = lax.axis_index("c")
    s = lax.axis_index("s")
    for r in range(8):
        for q in range(HH // 16):
            z8[r, pl.ds(q * 16, 16)] = jnp.zeros((16,), jnp.float32)
    zb = s * RPT
    pltpu.sync_copy(tbl_hbm.at[c, pl.ds(zb, RPT)], tbl_s.at[pl.ds(zb, RPT)])
    lax.fori_loop(
        0, RPT // 8,
        lambda j, _: (pltpu.sync_copy(z8, acc.at[pl.ds(zb + j * 8, 8)]), _)[1],
        None)
    base = s * CPTF
    pltpu.sync_copy(e3d_hbm.at[0, pl.ds(base, CPTF)], src_v.at[pl.ds(0, CPTF)])
    pltpu.sync_copy(e3d_hbm.at[1, pl.ds(base, CPTF)], dst_v.at[pl.ds(0, CPTF)])

    @pl.when(s < NTAILF)
    def _tail_load():
        pltpu.sync_copy(e3d_hbm.at[0, pl.ds(NS * CPTF + s, 1)],
                        src_v.at[pl.ds(CPTF, 1)])
        pltpu.sync_copy(e3d_hbm.at[1, pl.ds(NS * CPTF + s, 1)],
                        dst_v.at[pl.ds(CPTF, 1)])

    plsc.subcore_barrier()

    bufs = ((rows_a, sem_a, sem_sa), (rows_b, sem_b, sem_sb))
    pltpu.async_copy(tbl_s.at[src_v.at[0]], rows_a, sem_a)

    def group(g, _):
        for b in range(2):
            j = 2 * g + b
            buf, sem, sem_s = bufs[b]
            nbuf, nsem, nsem_s = bufs[1 - b]

            def _wait_prev_scatter():
                pltpu.make_async_copy(nbuf, acc.at[dst_v.at[j - 1]],
                                      nsem_s).wait()

            if b == 0:
                pl.when(j > 0)(_wait_prev_scatter)
                pltpu.async_copy(tbl_s.at[src_v.at[j + 1]], nbuf, nsem)
            else:
                _wait_prev_scatter()

                @pl.when(j + 1 < CPTF)
                def _start():
                    pltpu.async_copy(tbl_s.at[src_v.at[j + 1]], nbuf, nsem)
            pltpu.make_async_copy(tbl_s.at[src_v.at[j]], buf, sem).wait()
            pltpu.async_copy(buf, acc.at[dst_v.at[j]], sem_s, add=True)
        return _

    lax.fori_loop(0, CPTF // 2, group, None)
    pltpu.make_async_copy(rows_b, acc.at[dst_v.at[CPTF - 1]], sem_sb).wait()

    @pl.when(s < NTAILF)
    def _tail_chunk():
        pltpu.async_copy(tbl_s.at[src_v.at[CPTF]], rows_b, sem_b).wait()
        pltpu.sync_copy(rows_b, acc.at[dst_v.at[CPTF]], add=True)

    plsc.subcore_barrier()
    pltpu.sync_copy(acc.at[pl.ds(s * RPT, RPT)],
                    out_hbm.at[c, pl.ds(s * RPT, RPT)])


def _tc0_body(x_ref, w1_ref, m_ref):
    m_ref[...] = jnp.dot(x_ref[...], w1_ref[...],
                         preferred_element_type=jnp.float32)


def _split_store(t_ref, v):
    t_ref[0, 0:N, :] = v[:, 0:HH]
    t_ref[1, 0:N, :] = v[:, HH:H]
    t_ref[0, N:ACC_ROWS, :] = jnp.zeros((ACC_ROWS - N, HH), jnp.float32)
    t_ref[1, N:ACC_ROWS, :] = jnp.zeros((ACC_ROWS - N, HH), jnp.float32)


def _merge(a_ref, t_ref=None):
    agg = jnp.concatenate([a_ref[0, 0:N, :], a_ref[1, 0:N, :]], axis=1)
    if t_ref is None:
        return agg
    return agg + jnp.concatenate([t_ref[0, 0:N, :], t_ref[1, 0:N, :]], axis=1)


def _tc1_body(m_ref, degp_ref, t1_ref, dinv_ref):
    deg = degp_ref[0:N] + degp_ref[ACC_ROWS:ACC_ROWS + N] + 1.0
    dinv = lax.rsqrt(deg)
    _split_store(t1_ref, m_ref[...] * dinv[:, None])
    dinv_ref[...] = dinv[:, None]


def _tc2_body(aggp_ref, t1_ref, dinv_ref, b1_ref, w2_ref, t2_ref):
    dinv = dinv_ref[...]
    h1 = jax.nn.relu(dinv * _merge(aggp_ref, t1_ref) + b1_ref[...])
    m2 = jnp.dot(h1, w2_ref[...], preferred_element_type=jnp.float32)
    _split_store(t2_ref, m2 * dinv)


def _tc3_body(aggp_ref, t2_ref, dinv_ref, b2_ref, batch_ref, cfg_ref,
              fc1w_ref, fc1b_ref, fc2w_ref, fc2b_ref, out_ref):
    h2 = jax.nn.relu(dinv_ref[...] * _merge(aggp_ref, t2_ref) + b2_ref[...])
    gids = lax.broadcasted_iota(jnp.int32, (N, G), 1)
    p = (batch_ref[...] == gids).astype(jnp.float32)
    sums = lax.dot_general(p, h2, (((0,), (0,)), ((), ())),
                           preferred_element_type=jnp.float32)
    cnt = jnp.sum(p, axis=0)
    pooled = sums / jnp.maximum(cnt, 1.0)[:, None]
    z = jnp.concatenate([pooled, cfg_ref[...]], axis=1)
    z = jax.nn.relu(jnp.dot(z, fc1w_ref[...],
                            preferred_element_type=jnp.float32) + fc1b_ref[...])
    out_ref[...] = jnp.dot(z, fc2w_ref[...],
                           preferred_element_type=jnp.float32) + fc2b_ref[...]


def _vmem_call(body, n_in, out_shapes):
    return pl.pallas_call(
        body,
        in_specs=[pl.BlockSpec(memory_space=pltpu.VMEM) for _ in range(n_in)],
        out_specs=tuple(pl.BlockSpec(memory_space=pltpu.VMEM)
                        for _ in out_shapes) if len(out_shapes) > 1
        else pl.BlockSpec(memory_space=pltpu.VMEM),
        out_shape=out_shapes if len(out_shapes) > 1 else out_shapes[0],
    )


def kernel(x, edge_index, config, batch, W1, b1, W2, b2,
           fc1_W, fc1_b, fc2_W, fc2_b):
    e3d = edge_index.astype(jnp.int32).reshape(2, NCHUNK, CH)

    degp = _sc_degree(e3d)
    m1 = _vmem_call(
        _tc0_body, 2,
        (jax.ShapeDtypeStruct((N, H), jnp.float32),),
    )(x, W1)

    t1, dinv = _vmem_call(
        _tc1_body, 2,
        (jax.ShapeDtypeStruct((NC, ACC_ROWS, HH), jnp.float32),
         jax.ShapeDtypeStruct((N, 1), jnp.float32)),
    )(m1, degp)

    aggp1 = _sc_aggregate(e3d, t1)

    t2 = _vmem_call(
        _tc2_body, 5,
        (jax.ShapeDtypeStruct((NC, ACC_ROWS, HH), jnp.float32),),
    )(aggp1, t1, dinv, b1.reshape(1, H), W2)

    aggp2 = _sc_aggregate(e3d, t2)

    out = _vmem_call(
        _tc3_body, 10,
        (jax.ShapeDtypeStruct((G, 1), jnp.float32),),
    )(aggp2, t2, dinv, b2.reshape(1, H), batch.astype(jnp.int32).reshape(N, 1),
      config, fc1_W, fc1_b.reshape(1, H), fc2_W, fc2_b.reshape(1, 1))
    return out
